# baseline (speedup 1.0000x reference)
import functools

import jax
import jax.numpy as jnp
from jax import lax
from jax.experimental import pallas as pl
from jax.experimental.pallas import tpu as pltpu
from jax.experimental.pallas import tpu_sc as plsc

_K0 = 928981903
_K1 = 3453687069

_NC = 2
_NS = 16
_NW = _NC * _NS
_GRP = 128


def _keep_scale(idx_i32):
  k0 = jnp.uint32(_K0)
  k1 = jnp.uint32(_K1)
  k2 = k0 ^ k1 ^ jnp.uint32(0x1BD11BDA)
  ks = (k0, k1, k2)
  rot = ((13, 15, 26, 6), (17, 29, 16, 24))

  x0 = jnp.zeros((16,), jnp.uint32) + k0
  x1 = idx_i32.astype(jnp.uint32) + k1
  for i in range(5):
    for r in rot[i % 2]:
      x0 = x0 + x1
      x1 = (x1 << jnp.uint32(r)) | (x1 >> jnp.uint32(32 - r))
      x1 = x1 ^ x0
    x0 = x0 + ks[(i + 1) % 3]
    x1 = x1 + ks[(i + 2) % 3] + jnp.uint32(i + 1)
  bits = x0 ^ x1
  keep = (bits >> jnp.uint32(9)) <= jnp.uint32(6710886)
  return jnp.where(keep, jnp.float32(1.25), jnp.float32(0.0))


_TNBUF = 4


_DP = 33


@functools.lru_cache(maxsize=None)
def _make_relayout(v, d):
  assert d == 32
  full = v // _GRP
  main = full - full % _NW
  per_w = main // _NW
  assert per_w % _TNBUF == 0
  kmax = per_w // _TNBUF
  rem = full - main
  tail = v - full * _GRP
  blk = _GRP * d
  mesh = plsc.VectorSubcoreMesh(core_axis_name="c", subcore_axis_name="s")

  def transpose_block(src, pad, dst, ncols):
    lane_idx = lax.iota(jnp.int32, 16) * _DP

    def cols(cb, c):
      for dd in range(d):
        vals = src[dd, pl.ds(cb * 16, 16)]
        plsc.store_scatter(pad, [lane_idx + (cb * (16 * _DP) + dd)], vals)
      return c

    lax.fori_loop(0, ncols // 16, cols, 0)

    def repack(c8, c):
      for u in range(8):
        col = c8 * 8 + u
        dst[pl.ds(col * d, 16)] = pad[pl.ds(col * _DP, 16)]
        dst[pl.ds(col * d + 16, 16)] = pad[pl.ds(col * _DP + 16, 16)]
      return c

    lax.fori_loop(0, ncols // 8, repack, 0)

  @functools.partial(
      pl.kernel,
      mesh=mesh,
      out_type=jax.ShapeDtypeStruct((v * d,), jnp.float32),
      compiler_params=pltpu.CompilerParams(needs_layout_passes=False),
      scratch_types=(
          [pltpu.VMEM((_TNBUF, d, _GRP), jnp.float32),
           pltpu.VMEM((_GRP * _DP,), jnp.float32)]
          + [pltpu.VMEM((blk,), jnp.float32)] * _TNBUF
          + [pltpu.SemaphoreType.DMA((_TNBUF,)),
             pltpu.SemaphoreType.DMA((_TNBUF,))]
      ),
  )
  def relayout(wt_hbm, wtail_hbm, wlin_hbm, inb, pad, *rest):
    outb = rest[:_TNBUF]
    gsem, ssem = rest[_TNBUF], rest[_TNBUF + 1]
    wid = lax.axis_index("s") * _NC + lax.axis_index("c")
    c0 = wid * per_w

    for b in range(_TNBUF):
      pltpu.async_copy(
          wt_hbm.at[:, pl.ds((c0 + b) * _GRP, _GRP)], inb.at[b], gsem.at[b])

    def step(k, carry):
      for b in range(_TNBUF):
        c = c0 + k * _TNBUF + b
        pltpu.make_async_copy(
            wt_hbm.at[:, pl.ds(c * _GRP, _GRP)], inb.at[b], gsem.at[b]).wait()

        @pl.when(k > 0)
        def _():
          pltpu.make_async_copy(
              outb[b], wlin_hbm.at[pl.ds(c * blk, blk)], ssem.at[b]).wait()

        transpose_block(inb.at[b], pad, outb[b], _GRP)
        pltpu.async_copy(
            outb[b], wlin_hbm.at[pl.ds(c * blk, blk)], ssem.at[b])

        @pl.when(k < kmax - 1)
        def _():
          pltpu.async_copy(
              wt_hbm.at[:, pl.ds((c + _TNBUF) * _GRP, _GRP)], inb.at[b],
              gsem.at[b])

      return carry

    lax.fori_loop(0, kmax, step, 0)
    for b in range(_TNBUF):
      pltpu.make_async_copy(
          outb[b], wlin_hbm.at[pl.ds(0, blk)], ssem.at[b]).wait()

    if rem:
      @pl.when(wid < rem)
      def _():
        c = main + wid
        pltpu.sync_copy(wt_hbm.at[:, pl.ds(c * _GRP, _GRP)], inb.at[0])
        transpose_block(inb.at[0], pad, outb[0], _GRP)
        pltpu.sync_copy(outb[0], wlin_hbm.at[pl.ds(c * blk, blk)])

    if tail:
      @pl.when(wid == _NW - 1)
      def _():
        pltpu.sync_copy(wtail_hbm, outb[0].at[pl.ds(0, tail * d)])
        pltpu.sync_copy(outb[0].at[pl.ds(0, tail * d)],
                        wlin_hbm.at[pl.ds(full * blk, tail * d)])

  return relayout


_NBUF = 10


@functools.lru_cache(maxsize=None)
def _make_sc_lookup(n_l, n_b, v, d):
  assert d == 32 and n_b % _GRP == 0
  bpl = n_b // _GRP
  n_flat = n_l * n_b
  gpw = n_flat // (_NW * _GRP)
  assert gpw % _NBUF == 0
  kmax = gpw // _NBUF
  npw = gpw * _GRP
  mesh = plsc.VectorSubcoreMesh(core_axis_name="c", subcore_axis_name="s")

  @functools.partial(
      pl.kernel,
      mesh=mesh,
      out_type=jax.ShapeDtypeStruct((n_l * d * n_b,), jnp.float32),
      compiler_params=pltpu.CompilerParams(
          use_tc_tiling_on_sc=False, needs_layout_passes=False),
      scratch_types=(
          [pltpu.VMEM((npw,), jnp.int32),
           pltpu.VMEM((npw,), jnp.float32),
           pltpu.VMEM((_GRP * _DP,), jnp.float32)]
          + [pltpu.VMEM((_GRP, d), jnp.float32)] * _NBUF
          + [pltpu.VMEM((d * _GRP,), jnp.float32)] * _NBUF
          + [pltpu.SemaphoreType.DMA((_NBUF,)),
             pltpu.SemaphoreType.DMA((_NBUF,))]
      ),
  )
  def lookup(x_hbm, w_hbm, out_hbm, idx_v, scale_v, jbuf, *rest2):
    ibuf = rest2[:_NBUF]
    tbuf = rest2[_NBUF:2 * _NBUF]
    gsem, ssem = rest2[2 * _NBUF], rest2[2 * _NBUF + 1]
    wid = lax.axis_index("s") * _NC + lax.axis_index("c")
    g0 = wid * gpw
    pltpu.sync_copy(x_hbm.at[pl.ds(g0 * _GRP, npw)], idx_v)

    for b in range(_NBUF):
      pltpu.async_copy(
          w_hbm.at[idx_v.at[pl.ds(b * _GRP, _GRP)]], ibuf[b], gsem.at[b])

    def scales(t, c):
      scale_v[pl.ds(t * 16, 16)] = _keep_scale(idx_v[pl.ds(t * 16, 16)])
      return c

    lax.fori_loop(0, npw // 16, scales, 0)

    def store_wait(b):
      pltpu.make_async_copy(
          out_hbm.at[pl.ds(0, d * _GRP)], tbuf[b], ssem.at[b]).wait()

    def step(k, carry):
      for b in range(_NBUF):
        g = k * _NBUF + b
        pltpu.make_async_copy(
            w_hbm.at[idx_v.at[pl.ds(g * _GRP, _GRP)]], ibuf[b],
            gsem.at[b]).wait()
        @pl.when(k > 0)
        def _():
          store_wait(b)

        def stage8(r8, c):
          for u in range(8):
            r = r8 * 8 + u
            jbuf[pl.ds(r * _DP, 16)] = ibuf[b][r, pl.ds(0, 16)]
            jbuf[pl.ds(r * _DP + 16, 16)] = ibuf[b][r, pl.ds(16, 16)]
          return c

        lax.fori_loop(0, _GRP // 8, stage8, 0)

        lane_idx = lax.iota(jnp.int32, 16) * _DP

        def blk16(rr, c):
          base = rr * 16
          sv = scale_v[pl.ds(g * _GRP + base, 16)]
          ridx = lane_idx + base * _DP
          for dd in range(d):
            vals = plsc.load_gather(jbuf, [ridx + dd])
            tbuf[b][pl.ds(dd * _GRP + base, 16)] = vals * sv
          return c

        lax.fori_loop(0, _GRP // 16, blk16, 0)
        lo = ((g0 + g) // bpl) * (d * n_b) + ((g0 + g) % bpl) * _GRP
        for dd in range(d):
          pltpu.async_copy(
              tbuf[b].at[pl.ds(dd * _GRP, _GRP)],
              out_hbm.at[pl.ds(lo + dd * n_b, _GRP)], ssem.at[b])

        @pl.when(k < kmax - 1)
        def _():
          pltpu.async_copy(
              w_hbm.at[idx_v.at[pl.ds((g + _NBUF) * _GRP, _GRP)]], ibuf[b],
              gsem.at[b])

      return carry

    lax.fori_loop(0, kmax, step, 0)

    for b in range(_NBUF):
      store_wait(b)

  return lookup


def kernel(x, W):
  b, l = x.shape
  v, d = W.shape
  tail0 = (v // _GRP) * _GRP
  wtail = W[tail0:, :].reshape((v - tail0) * d)
  w_lin = _make_relayout(v, d)(W.T, wtail)
  w_rm = w_lin.reshape(v, d)
  xt = x.T.reshape(b * l)
  y = _make_sc_lookup(l, b, v, d)(xt, w_rm)
  return jnp.transpose(y.reshape(l, d, b), (2, 0, 1))


# speedup vs baseline: 1.0141x; 1.0141x over previous
import functools

import jax
import jax.numpy as jnp
from jax import lax
from jax.experimental import pallas as pl
from jax.experimental.pallas import tpu as pltpu
from jax.experimental.pallas import tpu_sc as plsc

_K0 = 928981903
_K1 = 3453687069

_NC = 2
_NS = 16
_NW = _NC * _NS
_GRP = 128


def _keep_scale(idx_i32):
  k0 = jnp.uint32(_K0)
  k1 = jnp.uint32(_K1)
  k2 = k0 ^ k1 ^ jnp.uint32(0x1BD11BDA)
  ks = (k0, k1, k2)
  rot = ((13, 15, 26, 6), (17, 29, 16, 24))

  x0 = jnp.zeros((16,), jnp.uint32) + k0
  x1 = idx_i32.astype(jnp.uint32) + k1
  for i in range(5):
    for r in rot[i % 2]:
      x0 = x0 + x1
      x1 = (x1 << jnp.uint32(r)) | (x1 >> jnp.uint32(32 - r))
      x1 = x1 ^ x0
    x0 = x0 + ks[(i + 1) % 3]
    x1 = x1 + ks[(i + 2) % 3] + jnp.uint32(i + 1)
  bits = x0 ^ x1
  keep = (bits >> jnp.uint32(9)) <= jnp.uint32(6710886)
  return jnp.where(keep, jnp.float32(1.25), jnp.float32(0.0))


_TNBUF = 4


_DP = 33


@functools.lru_cache(maxsize=None)
def _make_relayout(v, d):
  assert d == 32
  full = v // _GRP
  main = full - full % _NW
  per_w = main // _NW
  assert per_w % _TNBUF == 0
  kmax = per_w // _TNBUF
  rem = full - main
  tail = v - full * _GRP
  blk = _GRP * d
  mesh = plsc.VectorSubcoreMesh(core_axis_name="c", subcore_axis_name="s")

  def transpose_block(src, pad, dst, ncols):
    lane_idx = lax.iota(jnp.int32, 16) * _DP

    def cols(cb, c):
      for dd in range(d):
        vals = src[dd, pl.ds(cb * 16, 16)]
        plsc.store_scatter(pad, [lane_idx + (cb * (16 * _DP) + dd)], vals)
      return c

    lax.fori_loop(0, ncols // 16, cols, 0)

    def repack(c8, c):
      for u in range(8):
        col = c8 * 8 + u
        dst[pl.ds(col * d, 16)] = pad[pl.ds(col * _DP, 16)]
        dst[pl.ds(col * d + 16, 16)] = pad[pl.ds(col * _DP + 16, 16)]
      return c

    lax.fori_loop(0, ncols // 8, repack, 0)

  @functools.partial(
      pl.kernel,
      mesh=mesh,
      out_type=jax.ShapeDtypeStruct((v * d,), jnp.float32),
      compiler_params=pltpu.CompilerParams(needs_layout_passes=False),
      scratch_types=(
          [pltpu.VMEM((_TNBUF, d, _GRP), jnp.float32),
           pltpu.VMEM((_GRP * _DP,), jnp.float32)]
          + [pltpu.VMEM((blk,), jnp.float32)] * _TNBUF
          + [pltpu.SemaphoreType.DMA((_TNBUF,)),
             pltpu.SemaphoreType.DMA((_TNBUF,))]
      ),
  )
  def relayout(wt_hbm, wtail_hbm, wlin_hbm, inb, pad, *rest):
    outb = rest[:_TNBUF]
    gsem, ssem = rest[_TNBUF], rest[_TNBUF + 1]
    wid = lax.axis_index("s") * _NC + lax.axis_index("c")
    c0 = wid * per_w

    for b in range(_TNBUF):
      pltpu.async_copy(
          wt_hbm.at[:, pl.ds((c0 + b) * _GRP, _GRP)], inb.at[b], gsem.at[b])

    def step(k, carry):
      for b in range(_TNBUF):
        c = c0 + k * _TNBUF + b
        pltpu.make_async_copy(
            wt_hbm.at[:, pl.ds(c * _GRP, _GRP)], inb.at[b], gsem.at[b]).wait()

        @pl.when(k > 0)
        def _():
          pltpu.make_async_copy(
              outb[b], wlin_hbm.at[pl.ds(c * blk, blk)], ssem.at[b]).wait()

        transpose_block(inb.at[b], pad, outb[b], _GRP)
        pltpu.async_copy(
            outb[b], wlin_hbm.at[pl.ds(c * blk, blk)], ssem.at[b])

        @pl.when(k < kmax - 1)
        def _():
          pltpu.async_copy(
              wt_hbm.at[:, pl.ds((c + _TNBUF) * _GRP, _GRP)], inb.at[b],
              gsem.at[b])

      return carry

    lax.fori_loop(0, kmax, step, 0)
    for b in range(_TNBUF):
      pltpu.make_async_copy(
          outb[b], wlin_hbm.at[pl.ds(0, blk)], ssem.at[b]).wait()

    if rem:
      @pl.when(wid < rem)
      def _():
        c = main + wid
        pltpu.sync_copy(wt_hbm.at[:, pl.ds(c * _GRP, _GRP)], inb.at[0])
        transpose_block(inb.at[0], pad, outb[0], _GRP)
        pltpu.sync_copy(outb[0], wlin_hbm.at[pl.ds(c * blk, blk)])

    if tail:
      @pl.when(wid == _NW - 1)
      def _():
        pltpu.sync_copy(wtail_hbm, outb[0].at[pl.ds(0, tail * d)])
        pltpu.sync_copy(outb[0].at[pl.ds(0, tail * d)],
                        wlin_hbm.at[pl.ds(full * blk, tail * d)])

  return relayout


_NBUF = 5


@functools.lru_cache(maxsize=None)
def _make_sc_lookup(n_l, n_b, v, d):
  assert d == 32 and n_b % _GRP == 0
  bpl = n_b // _GRP
  n_flat = n_l * n_b
  gpw = n_flat // (_NW * _GRP)
  assert gpw % _NBUF == 0
  kmax = gpw // _NBUF
  npw = gpw * _GRP
  mesh = plsc.VectorSubcoreMesh(core_axis_name="c", subcore_axis_name="s")

  @functools.partial(
      pl.kernel,
      mesh=mesh,
      out_type=jax.ShapeDtypeStruct((n_l * d * n_b,), jnp.float32),
      compiler_params=pltpu.CompilerParams(
          use_tc_tiling_on_sc=False, needs_layout_passes=False),
      scratch_types=(
          [pltpu.VMEM((npw,), jnp.int32),
           pltpu.VMEM((npw,), jnp.float32),
           pltpu.VMEM((_GRP * _DP,), jnp.float32)]
          + [pltpu.VMEM((_GRP, d), jnp.float32)] * _NBUF
          + [pltpu.VMEM((d * _GRP,), jnp.float32)] * _NBUF
          + [pltpu.SemaphoreType.DMA((_NBUF,)),
             pltpu.SemaphoreType.DMA((_NBUF,))]
      ),
  )
  def lookup(x_hbm, w_hbm, out_hbm, idx_v, scale_v, jbuf, *rest2):
    ibuf = rest2[:_NBUF]
    tbuf = rest2[_NBUF:2 * _NBUF]
    gsem, ssem = rest2[2 * _NBUF], rest2[2 * _NBUF + 1]
    wid = lax.axis_index("s") * _NC + lax.axis_index("c")
    g0 = wid * gpw
    pltpu.sync_copy(x_hbm.at[pl.ds(g0 * _GRP, npw)], idx_v)

    for b in range(_NBUF):
      pltpu.async_copy(
          w_hbm.at[idx_v.at[pl.ds(b * _GRP, _GRP)]], ibuf[b], gsem.at[b])

    def scales(t, c):
      scale_v[pl.ds(t * 16, 16)] = _keep_scale(idx_v[pl.ds(t * 16, 16)])
      return c

    lax.fori_loop(0, npw // 16, scales, 0)

    def store_wait(b):
      pltpu.make_async_copy(
          out_hbm.at[pl.ds(0, d * _GRP)], tbuf[b], ssem.at[b]).wait()

    def step(k, carry):
      for b in range(_NBUF):
        g = k * _NBUF + b
        pltpu.make_async_copy(
            w_hbm.at[idx_v.at[pl.ds(g * _GRP, _GRP)]], ibuf[b],
            gsem.at[b]).wait()
        @pl.when(k > 0)
        def _():
          store_wait(b)

        def stage8(r8, c):
          for u in range(8):
            r = r8 * 8 + u
            jbuf[pl.ds(r * _DP, 16)] = ibuf[b][r, pl.ds(0, 16)]
            jbuf[pl.ds(r * _DP + 16, 16)] = ibuf[b][r, pl.ds(16, 16)]
          return c

        lax.fori_loop(0, _GRP // 8, stage8, 0)

        lane_idx = lax.iota(jnp.int32, 16) * _DP

        def blk16(rr, c):
          base = rr * 16
          sv = scale_v[pl.ds(g * _GRP + base, 16)]
          ridx = lane_idx + base * _DP
          for dd in range(d):
            vals = plsc.load_gather(jbuf, [ridx + dd])
            tbuf[b][pl.ds(dd * _GRP + base, 16)] = vals * sv
          return c

        lax.fori_loop(0, _GRP // 16, blk16, 0)
        lo = ((g0 + g) // bpl) * (d * n_b) + ((g0 + g) % bpl) * _GRP
        for dd in range(d):
          pltpu.async_copy(
              tbuf[b].at[pl.ds(dd * _GRP, _GRP)],
              out_hbm.at[pl.ds(lo + dd * n_b, _GRP)], ssem.at[b])

        @pl.when(k < kmax - 1)
        def _():
          pltpu.async_copy(
              w_hbm.at[idx_v.at[pl.ds((g + _NBUF) * _GRP, _GRP)]], ibuf[b],
              gsem.at[b])

      return carry

    lax.fori_loop(0, kmax, step, 0)

    for b in range(_NBUF):
      store_wait(b)

  return lookup


def kernel(x, W):
  b, l = x.shape
  v, d = W.shape
  tail0 = (v // _GRP) * _GRP
  wtail = W[tail0:, :].reshape((v - tail0) * d)
  w_lin = _make_relayout(v, d)(W.T, wtail)
  w_rm = w_lin.reshape(v, d)
  xt = x.T.reshape(b * l)
  y = _make_sc_lookup(l, b, v, d)(xt, w_rm)
  return jnp.transpose(y.reshape(l, d, b), (2, 0, 1))
